# baseline (speedup 1.0000x reference)
import functools

import jax
import jax.numpy as jnp
from jax import lax
from jax.experimental import pallas as pl
from jax.experimental.pallas import tpu as pltpu
from jax.experimental.pallas import tpu_sc as plsc

NC = 2
NS = 16
NW = NC * NS
L = 16
CH = 128


def _sc_hist(edge_index, n_pad, base, rem):
    npt = n_pad // NS
    mesh = plsc.VectorSubcoreMesh(core_axis_name="c", subcore_axis_name="s")

    @functools.partial(
        pl.kernel,
        out_type=jax.ShapeDtypeStruct((NC, n_pad), jnp.float32),
        mesh=mesh,
        scratch_types=[
            pltpu.VMEM((base * CH,), jnp.int32),
            pltpu.VMEM((base + 1, CH), jnp.int32),
            pltpu.VMEM((CH,), jnp.float32),
            pltpu.VMEM((npt,), jnp.float32),
            pltpu.VMEM_SHARED((n_pad,), jnp.float32),
            pltpu.SemaphoreType.DMA,
        ],
    )
    def hist_kernel(ei_hbm, deg_hbm, dst1, dst2, ones_v, zer_v, hist_sh,
                    sd0):
        cid = lax.axis_index("c")
        sid = lax.axis_index("s")
        wid = cid * NS + sid
        row0 = wid * base

        pltpu.async_copy(ei_hbm.at[1, pl.ds(row0 * CH, base * CH)], dst1,
                         sd0)

        one = jnp.ones((L,), jnp.float32)
        zero = jnp.zeros((L,), jnp.float32)
        for k in range(CH // L):
            ones_v[pl.ds(k * L, L)] = one

        @pl.loop(0, npt // L)
        def _(i):
            zer_v[pl.ds(i * L, L)] = zero

        pltpu.sync_copy(zer_v, hist_sh.at[pl.ds(sid * npt, npt)])
        pltpu.make_async_copy(ei_hbm.at[1, pl.ds(row0 * CH, base * CH)],
                              dst1, sd0).wait()

        @pl.loop(0, base)
        def _(r):
            for k in range(CH // L):
                dst2[r, pl.ds(k * L, L)] = dst1[pl.ds(r * CH + k * L, L)]

        if rem:
            @pl.when(wid < rem)
            def _():
                pltpu.sync_copy(
                    ei_hbm.at[1, pl.ds((NW * base + wid) * CH, CH)],
                    dst1.at[pl.ds(0, CH)])
                for k in range(CH // L):
                    dst2[base, pl.ds(k * L, L)] = dst1[pl.ds(k * L, L)]

        plsc.subcore_barrier()

        @pl.loop(0, base)
        def _(c):
            pltpu.async_copy(ones_v, hist_sh.at[dst2.at[c]], sd0,
                             add=True)

        if rem:
            @pl.when(wid < rem)
            def _():
                pltpu.async_copy(ones_v, hist_sh.at[dst2.at[base]], sd0,
                                 add=True)

        @pl.loop(0, base)
        def _(c):
            pltpu.make_async_copy(ones_v, hist_sh.at[dst2.at[c]],
                                  sd0).wait()

        if rem:
            @pl.when(wid < rem)
            def _():
                pltpu.make_async_copy(ones_v, hist_sh.at[dst2.at[base]],
                                      sd0).wait()

        plsc.subcore_barrier()
        pltpu.sync_copy(hist_sh.at[pl.ds(sid * npt, npt)],
                        deg_hbm.at[cid, pl.ds(sid * npt, npt)])

    return hist_kernel(edge_index)


def _sc_gather_scatter(g, edge_index, n_pad, d, base, rem):
    npt = n_pad // NS
    nloop = (base - 2) // 4 * 4
    mesh = plsc.VectorSubcoreMesh(core_axis_name="c", subcore_axis_name="s")

    @functools.partial(
        pl.kernel,
        out_type=jax.ShapeDtypeStruct((NC, n_pad, d), jnp.float32),
        mesh=mesh,
        scratch_types=[
            pltpu.VMEM((4, 2, CH), jnp.int32),
            pltpu.VMEM((2, CH, d), jnp.float32),
            pltpu.VMEM_SHARED((n_pad, d), jnp.float32),
            pltpu.SemaphoreType.DMA,
            pltpu.SemaphoreType.DMA,
            pltpu.SemaphoreType.DMA,
            pltpu.SemaphoreType.DMA,
            pltpu.SemaphoreType.DMA,
            pltpu.SemaphoreType.DMA,
        ],
    )
    def gs_kernel(g_hbm, ei_hbm, acc_hbm, idx_v, rows_v, acc_sh,
                  si0, si1, si2, si3, sg0, sg1):
        cid = lax.axis_index("c")
        sid = lax.axis_index("s")
        wid = cid * NS + sid
        semi = [si0, si1, si2, si3]
        semg = [sg0, sg1]

        def prefetch(row, slot):
            pltpu.async_copy(ei_hbm.at[0, pl.ds(row * CH, CH)],
                             idx_v.at[slot, 0], semi[slot])
            pltpu.async_copy(ei_hbm.at[1, pl.ds(row * CH, CH)],
                             idx_v.at[slot, 1], semi[slot])

        def fire_gather(row, slot, b):
            pltpu.make_async_copy(ei_hbm.at[0, pl.ds(row * CH, CH)],
                                  idx_v.at[slot, 0], semi[slot]).wait()
            pltpu.make_async_copy(ei_hbm.at[1, pl.ds(row * CH, CH)],
                                  idx_v.at[slot, 1], semi[slot]).wait()
            pltpu.async_copy(g_hbm.at[idx_v.at[slot, 0]], rows_v.at[b],
                             semg[b])

        def consume(slot, b):
            pltpu.make_async_copy(g_hbm.at[idx_v.at[slot, 0]], rows_v.at[b],
                                  semg[b]).wait()
            pltpu.sync_copy(rows_v.at[b], acc_sh.at[idx_v.at[slot, 1]],
                            add=True)

        zero = jnp.zeros((L,), jnp.float32)

        @pl.loop(0, CH)
        def _(r):
            for k in range(d // L):
                rows_v[0, r, pl.ds(k * L, L)] = zero

        for t in range(npt // CH):
            pltpu.sync_copy(rows_v.at[0],
                            acc_sh.at[pl.ds(sid * npt + t * CH, CH)])
        plsc.subcore_barrier()

        row0 = wid * base

        for k in range(4):
            prefetch(row0 + k, k)
        fire_gather(row0, 0, 0)

        @pl.loop(0, nloop, step=4)
        def _(c):
            fire_gather(row0 + c + 1, 1, 1)
            consume(0, 0)
            prefetch(row0 + c + 4, 0)
            fire_gather(row0 + c + 2, 2, 0)
            consume(1, 1)
            prefetch(row0 + c + 5, 1)
            fire_gather(row0 + c + 3, 3, 1)
            consume(2, 0)

            @pl.when(c + 6 < base)
            def _():
                prefetch(row0 + c + 6, 2)

            fire_gather(row0 + c + 4, 0, 0)
            consume(3, 1)

            @pl.when(c + 7 < base)
            def _():
                prefetch(row0 + c + 7, 3)

        diff = base - nloop
        if diff >= 2:
            fire_gather(row0 + nloop + 1, 1, 1)
        consume(0, 0)
        if diff >= 3:
            fire_gather(row0 + nloop + 2, 2, 0)
        if diff >= 2:
            consume(1, 1)
        if diff >= 4:
            fire_gather(row0 + nloop + 3, 3, 1)
        if diff >= 3:
            consume(2, 0)
        if diff >= 4:
            consume(3, 1)

        if rem:
            @pl.when(wid < rem)
            def _():
                erow = NW * base + wid
                prefetch(erow, 0)
                fire_gather(erow, 0, 0)
                consume(0, 0)

        plsc.subcore_barrier()
        pltpu.sync_copy(acc_sh.at[pl.ds(sid * npt, npt)],
                        acc_hbm.at[cid, pl.ds(sid * npt, npt)])

    return gs_kernel(g, edge_index)


def _tc_matmul(x, W, n_pad, blk):
    d = W.shape[0]

    def body(x_ref, w_ref, h_ref):
        h_ref[...] = jnp.dot(x_ref[...], w_ref[...],
                             preferred_element_type=jnp.float32)

    return pl.pallas_call(
        body,
        grid=(n_pad // blk,),
        in_specs=[
            pl.BlockSpec((blk, d), lambda i: (i, 0)),
            pl.BlockSpec((d, d), lambda i: (0, 0)),
        ],
        out_specs=pl.BlockSpec((blk, d), lambda i: (i, 0)),
        out_shape=jax.ShapeDtypeStruct((n_pad, d), jnp.float32),
    )(x, W)


def _dinv_col(deg_ref):
    deg = deg_ref[0:1, :] + deg_ref[1:2, :] + 1.0
    return jnp.transpose(lax.rsqrt(deg), (1, 0))


def _tc_scale(h, degp, n_pad, blk):
    d = h.shape[1]

    def body(h_ref, deg_ref, g_ref):
        g_ref[...] = h_ref[...] * _dinv_col(deg_ref)

    return pl.pallas_call(
        body,
        grid=(n_pad // blk,),
        in_specs=[
            pl.BlockSpec((blk, d), lambda i: (i, 0)),
            pl.BlockSpec((NC, blk), lambda i: (0, i)),
        ],
        out_specs=pl.BlockSpec((blk, d), lambda i: (i, 0)),
        out_shape=jax.ShapeDtypeStruct((n_pad, d), jnp.float32),
    )(h, degp)


def _tc_finalize(accp, g, degp, b2, n, n_pad, blk):
    d = g.shape[1]

    def body(acc_ref, g_ref, deg_ref, b_ref, o_ref):
        s = (acc_ref[0] + acc_ref[1] + g_ref[...]) * _dinv_col(deg_ref)
        o_ref[...] = jnp.maximum(s + b_ref[...], 0.0)

    return pl.pallas_call(
        body,
        grid=(n_pad // blk,),
        in_specs=[
            pl.BlockSpec((NC, blk, d), lambda i: (0, i, 0)),
            pl.BlockSpec((blk, d), lambda i: (i, 0)),
            pl.BlockSpec((NC, blk), lambda i: (0, i)),
            pl.BlockSpec((1, d), lambda i: (0, 0)),
        ],
        out_specs=pl.BlockSpec((blk, d), lambda i: (i, 0)),
        out_shape=jax.ShapeDtypeStruct((n, d), jnp.float32),
    )(accp, g, degp, b2)


def kernel(x, edge_index, W, b):
    n, d = x.shape
    e = edge_index.shape[1]
    n_pad = ((n + NS * CH - 1) // (NS * CH)) * (NS * CH)

    nrows = e // CH
    base = nrows // NW
    rem = nrows - base * NW

    degp = _sc_hist(edge_index, n_pad, base, rem)

    blk = 1024
    h = _tc_matmul(x, W, n_pad, blk)

    g = _tc_scale(h, degp, n_pad, blk)

    accp = _sc_gather_scatter(g, edge_index, n_pad, d, base, rem)

    return _tc_finalize(accp, g, degp, b.reshape(1, d), n, n_pad, blk)


# speedup vs baseline: 1.0396x; 1.0396x over previous
import functools

import jax
import jax.numpy as jnp
from jax import lax
from jax.experimental import pallas as pl
from jax.experimental.pallas import tpu as pltpu
from jax.experimental.pallas import tpu_sc as plsc

NC = 2
NS = 16
NW = NC * NS
L = 16
CH = 128


def _sc_hist(edge_index, n_pad, base, rem):
    npt = n_pad // NS
    mesh = plsc.VectorSubcoreMesh(core_axis_name="c", subcore_axis_name="s")

    @functools.partial(
        pl.kernel,
        out_type=jax.ShapeDtypeStruct((NC, n_pad), jnp.float32),
        mesh=mesh,
        scratch_types=[
            pltpu.VMEM((base * CH,), jnp.int32),
            pltpu.VMEM((base + 1, CH), jnp.int32),
            pltpu.VMEM((CH,), jnp.float32),
            pltpu.VMEM((npt,), jnp.float32),
            pltpu.VMEM_SHARED((n_pad,), jnp.float32),
            pltpu.SemaphoreType.DMA,
        ],
    )
    def hist_kernel(ei_hbm, deg_hbm, dst1, dst2, ones_v, zer_v, hist_sh,
                    sd0):
        cid = lax.axis_index("c")
        sid = lax.axis_index("s")
        wid = cid * NS + sid
        row0 = wid * base

        pltpu.async_copy(ei_hbm.at[1, pl.ds(row0 * CH, base * CH)], dst1,
                         sd0)

        one = jnp.ones((L,), jnp.float32)
        zero = jnp.zeros((L,), jnp.float32)
        for k in range(CH // L):
            ones_v[pl.ds(k * L, L)] = one

        @pl.loop(0, npt // L)
        def _(i):
            zer_v[pl.ds(i * L, L)] = zero

        pltpu.sync_copy(zer_v, hist_sh.at[pl.ds(sid * npt, npt)])
        pltpu.make_async_copy(ei_hbm.at[1, pl.ds(row0 * CH, base * CH)],
                              dst1, sd0).wait()

        @pl.loop(0, base)
        def _(r):
            for k in range(CH // L):
                dst2[r, pl.ds(k * L, L)] = dst1[pl.ds(r * CH + k * L, L)]

        if rem:
            @pl.when(wid < rem)
            def _():
                pltpu.sync_copy(
                    ei_hbm.at[1, pl.ds((NW * base + wid) * CH, CH)],
                    dst1.at[pl.ds(0, CH)])
                for k in range(CH // L):
                    dst2[base, pl.ds(k * L, L)] = dst1[pl.ds(k * L, L)]

        plsc.subcore_barrier()

        @pl.loop(0, base)
        def _(c):
            pltpu.async_copy(ones_v, hist_sh.at[dst2.at[c]], sd0,
                             add=True)

        if rem:
            @pl.when(wid < rem)
            def _():
                pltpu.async_copy(ones_v, hist_sh.at[dst2.at[base]], sd0,
                                 add=True)

        @pl.loop(0, base)
        def _(c):
            pltpu.make_async_copy(ones_v, hist_sh.at[dst2.at[c]],
                                  sd0).wait()

        if rem:
            @pl.when(wid < rem)
            def _():
                pltpu.make_async_copy(ones_v, hist_sh.at[dst2.at[base]],
                                      sd0).wait()

        plsc.subcore_barrier()
        pltpu.sync_copy(hist_sh.at[pl.ds(sid * npt, npt)],
                        deg_hbm.at[cid, pl.ds(sid * npt, npt)])

    return hist_kernel(edge_index)


def _sc_gather_scatter(g, edge_index, n_pad, d, base, rem):
    npt = n_pad // NS
    nloop = (base - 2) // 4 * 4
    mesh = plsc.VectorSubcoreMesh(core_axis_name="c", subcore_axis_name="s")

    @functools.partial(
        pl.kernel,
        out_type=jax.ShapeDtypeStruct((NC, n_pad, d), jnp.float32),
        mesh=mesh,
        scratch_types=[
            pltpu.VMEM((4, 2, CH), jnp.int32),
            pltpu.VMEM((2, CH, d), jnp.float32),
            pltpu.VMEM_SHARED((n_pad, d), jnp.float32),
            pltpu.SemaphoreType.DMA,
            pltpu.SemaphoreType.DMA,
            pltpu.SemaphoreType.DMA,
            pltpu.SemaphoreType.DMA,
            pltpu.SemaphoreType.DMA,
            pltpu.SemaphoreType.DMA,
        ],
    )
    def gs_kernel(g_hbm, ei_hbm, acc_hbm, idx_v, rows_v, acc_sh,
                  si0, si1, si2, si3, sg0, sg1):
        cid = lax.axis_index("c")
        sid = lax.axis_index("s")
        wid = cid * NS + sid
        semi = [si0, si1, si2, si3]
        semg = [sg0, sg1]

        def prefetch(row, slot):
            pltpu.async_copy(ei_hbm.at[0, pl.ds(row * CH, CH)],
                             idx_v.at[slot, 0], semi[slot])
            pltpu.async_copy(ei_hbm.at[1, pl.ds(row * CH, CH)],
                             idx_v.at[slot, 1], semi[slot])

        def fire_gather(row, slot, b):
            pltpu.make_async_copy(ei_hbm.at[0, pl.ds(row * CH, CH)],
                                  idx_v.at[slot, 0], semi[slot]).wait()
            pltpu.make_async_copy(ei_hbm.at[1, pl.ds(row * CH, CH)],
                                  idx_v.at[slot, 1], semi[slot]).wait()
            pltpu.async_copy(g_hbm.at[idx_v.at[slot, 0]], rows_v.at[b],
                             semg[b])

        def consume(slot, b):
            pltpu.make_async_copy(g_hbm.at[idx_v.at[slot, 0]], rows_v.at[b],
                                  semg[b]).wait()
            pltpu.sync_copy(rows_v.at[b], acc_sh.at[idx_v.at[slot, 1]],
                            add=True)

        zero = jnp.zeros((L,), jnp.float32)

        @pl.loop(0, CH)
        def _(r):
            for k in range(d // L):
                rows_v[0, r, pl.ds(k * L, L)] = zero

        for t in range(npt // CH):
            pltpu.sync_copy(rows_v.at[0],
                            acc_sh.at[pl.ds(sid * npt + t * CH, CH)])
        plsc.subcore_barrier()

        row0 = wid * base

        for k in range(4):
            prefetch(row0 + k, k)
        fire_gather(row0, 0, 0)

        @pl.loop(0, nloop, step=4)
        def _(c):
            fire_gather(row0 + c + 1, 1, 1)
            consume(0, 0)
            prefetch(row0 + c + 4, 0)
            fire_gather(row0 + c + 2, 2, 0)
            consume(1, 1)
            prefetch(row0 + c + 5, 1)
            fire_gather(row0 + c + 3, 3, 1)
            consume(2, 0)

            @pl.when(c + 6 < base)
            def _():
                prefetch(row0 + c + 6, 2)

            fire_gather(row0 + c + 4, 0, 0)
            consume(3, 1)

            @pl.when(c + 7 < base)
            def _():
                prefetch(row0 + c + 7, 3)

        diff = base - nloop
        if diff >= 2:
            fire_gather(row0 + nloop + 1, 1, 1)
        consume(0, 0)
        if diff >= 3:
            fire_gather(row0 + nloop + 2, 2, 0)
        if diff >= 2:
            consume(1, 1)
        if diff >= 4:
            fire_gather(row0 + nloop + 3, 3, 1)
        if diff >= 3:
            consume(2, 0)
        if diff >= 4:
            consume(3, 1)

        if rem:
            @pl.when(wid < rem)
            def _():
                erow = NW * base + wid
                prefetch(erow, 0)
                fire_gather(erow, 0, 0)
                consume(0, 0)

        plsc.subcore_barrier()
        pltpu.sync_copy(acc_sh.at[pl.ds(sid * npt, npt)],
                        acc_hbm.at[cid, pl.ds(sid * npt, npt)])

    return gs_kernel(g, edge_index)


def _tc_matmul(x, W, n_pad, blk):
    d = W.shape[0]

    def body(x_ref, w_ref, h_ref):
        h_ref[...] = jnp.dot(x_ref[...], w_ref[...],
                             preferred_element_type=jnp.float32)

    return pl.pallas_call(
        body,
        grid=(n_pad // blk,),
        in_specs=[
            pl.BlockSpec((blk, d), lambda i: (i, 0)),
            pl.BlockSpec((d, d), lambda i: (0, 0)),
        ],
        out_specs=pl.BlockSpec((blk, d), lambda i: (i, 0)),
        out_shape=jax.ShapeDtypeStruct((n_pad, d), jnp.float32),
    )(x, W)


def _dinv_col(deg_ref):
    deg = deg_ref[0:1, :] + deg_ref[1:2, :] + 1.0
    return jnp.transpose(lax.rsqrt(deg), (1, 0))


def _tc_scale(h, degp, n_pad, blk):
    d = h.shape[1]

    def body(h_ref, deg_ref, g_ref):
        g_ref[...] = h_ref[...] * _dinv_col(deg_ref)

    return pl.pallas_call(
        body,
        grid=(n_pad // blk,),
        in_specs=[
            pl.BlockSpec((blk, d), lambda i: (i, 0)),
            pl.BlockSpec((NC, blk), lambda i: (0, i)),
        ],
        out_specs=pl.BlockSpec((blk, d), lambda i: (i, 0)),
        out_shape=jax.ShapeDtypeStruct((n_pad, d), jnp.float32),
    )(h, degp)


def _tc_finalize(accp, g, degp, b2, n, n_pad, blk):
    d = g.shape[1]

    def body(acc_ref, g_ref, deg_ref, b_ref, o_ref):
        s = (acc_ref[0] + acc_ref[1] + g_ref[...]) * _dinv_col(deg_ref)
        o_ref[...] = jnp.maximum(s + b_ref[...], 0.0)

    return pl.pallas_call(
        body,
        grid=(n_pad // blk,),
        in_specs=[
            pl.BlockSpec((NC, blk, d), lambda i: (0, i, 0)),
            pl.BlockSpec((blk, d), lambda i: (i, 0)),
            pl.BlockSpec((NC, blk), lambda i: (0, i)),
            pl.BlockSpec((1, d), lambda i: (0, 0)),
        ],
        out_specs=pl.BlockSpec((blk, d), lambda i: (i, 0)),
        out_shape=jax.ShapeDtypeStruct((n, d), jnp.float32),
    )(accp, g, degp, b2)


def kernel(x, edge_index, W, b):
    n, d = x.shape
    e = edge_index.shape[1]
    n_pad = ((n + NS * CH - 1) // (NS * CH)) * (NS * CH)

    nrows = e // CH
    base = nrows // NW
    rem = nrows - base * NW

    degp = _sc_hist(edge_index, n_pad, base, rem)

    blk = 2560
    h = _tc_matmul(x, W, n_pad, blk)

    g = _tc_scale(h, degp, n_pad, blk)

    accp = _sc_gather_scatter(g, edge_index, n_pad, d, base, rem)

    return _tc_finalize(accp, g, degp, b.reshape(1, d), n, n_pad, blk)


# speedup vs baseline: 1.0458x; 1.0060x over previous
import functools

import jax
import jax.numpy as jnp
from jax import lax
from jax.experimental import pallas as pl
from jax.experimental.pallas import tpu as pltpu
from jax.experimental.pallas import tpu_sc as plsc

NC = 2
NS = 16
NW = NC * NS
L = 16
CH = 128


def _sc_hist(edge_index, n_pad, base, rem):
    npt = n_pad // NS
    mesh = plsc.VectorSubcoreMesh(core_axis_name="c", subcore_axis_name="s")

    @functools.partial(
        pl.kernel,
        out_type=jax.ShapeDtypeStruct((NC, n_pad), jnp.float32),
        mesh=mesh,
        scratch_types=[
            pltpu.VMEM((base * CH,), jnp.int32),
            pltpu.VMEM((base + 1, CH), jnp.int32),
            pltpu.VMEM((CH,), jnp.float32),
            pltpu.VMEM((npt,), jnp.float32),
            pltpu.VMEM_SHARED((n_pad,), jnp.float32),
            pltpu.SemaphoreType.DMA,
        ],
    )
    def hist_kernel(ei_hbm, deg_hbm, dst1, dst2, ones_v, zer_v, hist_sh,
                    sd0):
        cid = lax.axis_index("c")
        sid = lax.axis_index("s")
        wid = cid * NS + sid
        row0 = wid * base

        pltpu.async_copy(ei_hbm.at[1, pl.ds(row0 * CH, base * CH)], dst1,
                         sd0)

        one = jnp.ones((L,), jnp.float32)
        zero = jnp.zeros((L,), jnp.float32)
        for k in range(CH // L):
            ones_v[pl.ds(k * L, L)] = one

        @pl.loop(0, npt // L)
        def _(i):
            zer_v[pl.ds(i * L, L)] = zero

        pltpu.sync_copy(zer_v, hist_sh.at[pl.ds(sid * npt, npt)])
        pltpu.make_async_copy(ei_hbm.at[1, pl.ds(row0 * CH, base * CH)],
                              dst1, sd0).wait()

        @pl.loop(0, base)
        def _(r):
            for k in range(CH // L):
                dst2[r, pl.ds(k * L, L)] = dst1[pl.ds(r * CH + k * L, L)]

        if rem:
            @pl.when(wid < rem)
            def _():
                pltpu.sync_copy(
                    ei_hbm.at[1, pl.ds((NW * base + wid) * CH, CH)],
                    dst1.at[pl.ds(0, CH)])
                for k in range(CH // L):
                    dst2[base, pl.ds(k * L, L)] = dst1[pl.ds(k * L, L)]

        plsc.subcore_barrier()

        @pl.loop(0, base)
        def _(c):
            pltpu.async_copy(ones_v, hist_sh.at[dst2.at[c]], sd0,
                             add=True)

        if rem:
            @pl.when(wid < rem)
            def _():
                pltpu.async_copy(ones_v, hist_sh.at[dst2.at[base]], sd0,
                                 add=True)

        @pl.loop(0, base)
        def _(c):
            pltpu.make_async_copy(ones_v, hist_sh.at[dst2.at[c]],
                                  sd0).wait()

        if rem:
            @pl.when(wid < rem)
            def _():
                pltpu.make_async_copy(ones_v, hist_sh.at[dst2.at[base]],
                                      sd0).wait()

        plsc.subcore_barrier()
        pltpu.sync_copy(hist_sh.at[pl.ds(sid * npt, npt)],
                        deg_hbm.at[cid, pl.ds(sid * npt, npt)])

    return hist_kernel(edge_index)


def _sc_gather_scatter(g, edge_index, n_pad, d, base, rem):
    npt = n_pad // NS
    nloop = (base - 2) // 4 * 4
    mesh = plsc.VectorSubcoreMesh(core_axis_name="c", subcore_axis_name="s")

    @functools.partial(
        pl.kernel,
        out_type=jax.ShapeDtypeStruct((NC, n_pad, d), jnp.float32),
        mesh=mesh,
        scratch_types=[
            pltpu.VMEM((4, 2, CH), jnp.int32),
            pltpu.VMEM((2, CH, d), jnp.float32),
            pltpu.VMEM_SHARED((n_pad, d), jnp.float32),
            pltpu.SemaphoreType.DMA,
            pltpu.SemaphoreType.DMA,
            pltpu.SemaphoreType.DMA,
            pltpu.SemaphoreType.DMA,
            pltpu.SemaphoreType.DMA,
            pltpu.SemaphoreType.DMA,
        ],
    )
    def gs_kernel(g_hbm, ei_hbm, acc_hbm, idx_v, rows_v, acc_sh,
                  si0, si1, si2, si3, sg0, sg1):
        cid = lax.axis_index("c")
        sid = lax.axis_index("s")
        wid = cid * NS + sid
        semi = [si0, si1, si2, si3]
        semg = [sg0, sg1]

        def prefetch(row, slot):
            pltpu.async_copy(ei_hbm.at[0, pl.ds(row * CH, CH)],
                             idx_v.at[slot, 0], semi[slot])
            pltpu.async_copy(ei_hbm.at[1, pl.ds(row * CH, CH)],
                             idx_v.at[slot, 1], semi[slot])

        def fire_gather(row, slot, b):
            pltpu.make_async_copy(ei_hbm.at[0, pl.ds(row * CH, CH)],
                                  idx_v.at[slot, 0], semi[slot]).wait()
            pltpu.make_async_copy(ei_hbm.at[1, pl.ds(row * CH, CH)],
                                  idx_v.at[slot, 1], semi[slot]).wait()
            pltpu.async_copy(g_hbm.at[idx_v.at[slot, 0]], rows_v.at[b],
                             semg[b])

        def consume(slot, b):
            pltpu.make_async_copy(g_hbm.at[idx_v.at[slot, 0]], rows_v.at[b],
                                  semg[b]).wait()
            pltpu.sync_copy(rows_v.at[b], acc_sh.at[idx_v.at[slot, 1]],
                            add=True)

        zero = jnp.zeros((L,), jnp.float32)

        @pl.loop(0, CH)
        def _(r):
            for k in range(d // L):
                rows_v[0, r, pl.ds(k * L, L)] = zero

        for t in range(npt // CH):
            pltpu.sync_copy(rows_v.at[0],
                            acc_sh.at[pl.ds(sid * npt + t * CH, CH)])
        plsc.subcore_barrier()

        row0 = wid * base

        for k in range(4):
            prefetch(row0 + k, k)
        fire_gather(row0, 0, 0)

        @pl.loop(0, nloop, step=4)
        def _(c):
            fire_gather(row0 + c + 1, 1, 1)
            consume(0, 0)
            prefetch(row0 + c + 4, 0)
            fire_gather(row0 + c + 2, 2, 0)
            consume(1, 1)
            prefetch(row0 + c + 5, 1)
            fire_gather(row0 + c + 3, 3, 1)
            consume(2, 0)

            @pl.when(c + 6 < base)
            def _():
                prefetch(row0 + c + 6, 2)

            fire_gather(row0 + c + 4, 0, 0)
            consume(3, 1)

            @pl.when(c + 7 < base)
            def _():
                prefetch(row0 + c + 7, 3)

        diff = base - nloop
        if diff >= 2:
            fire_gather(row0 + nloop + 1, 1, 1)
        consume(0, 0)
        if diff >= 3:
            fire_gather(row0 + nloop + 2, 2, 0)
        if diff >= 2:
            consume(1, 1)
        if diff >= 4:
            fire_gather(row0 + nloop + 3, 3, 1)
        if diff >= 3:
            consume(2, 0)
        if diff >= 4:
            consume(3, 1)

        if rem:
            @pl.when(wid < rem)
            def _():
                erow = NW * base + wid
                prefetch(erow, 0)
                fire_gather(erow, 0, 0)
                consume(0, 0)

        plsc.subcore_barrier()
        pltpu.sync_copy(acc_sh.at[pl.ds(sid * npt, npt)],
                        acc_hbm.at[cid, pl.ds(sid * npt, npt)])

    return gs_kernel(g, edge_index)


def _tc_matmul(x, W, n_pad, blk):
    d = W.shape[0]

    def body(x_ref, w_ref, h_ref):
        h_ref[...] = jnp.dot(x_ref[...], w_ref[...],
                             preferred_element_type=jnp.float32)

    return pl.pallas_call(
        body,
        grid=(n_pad // blk,),
        in_specs=[
            pl.BlockSpec((blk, d), lambda i: (i, 0)),
            pl.BlockSpec((d, d), lambda i: (0, 0)),
        ],
        out_specs=pl.BlockSpec((blk, d), lambda i: (i, 0)),
        out_shape=jax.ShapeDtypeStruct((n_pad, d), jnp.float32),
    )(x, W)


def _dinv_col(deg_ref):
    deg = deg_ref[0:1, :] + deg_ref[1:2, :] + 1.0
    return jnp.transpose(lax.rsqrt(deg), (1, 0))


def _tc_scale(h, degp, n_pad, blk):
    d = h.shape[1]

    def body(h_ref, deg_ref, g_ref):
        g_ref[...] = h_ref[...] * _dinv_col(deg_ref)

    return pl.pallas_call(
        body,
        grid=(n_pad // blk,),
        in_specs=[
            pl.BlockSpec((blk, d), lambda i: (i, 0)),
            pl.BlockSpec((NC, blk), lambda i: (0, i)),
        ],
        out_specs=pl.BlockSpec((blk, d), lambda i: (i, 0)),
        out_shape=jax.ShapeDtypeStruct((n_pad, d), jnp.float32),
    )(h, degp)


def _tc_finalize(accp, g, degp, b2, n, n_pad, blk):
    d = g.shape[1]

    def body(acc_ref, g_ref, deg_ref, b_ref, o_ref):
        s = (acc_ref[0] + acc_ref[1] + g_ref[...]) * _dinv_col(deg_ref)
        o_ref[...] = jnp.maximum(s + b_ref[...], 0.0)

    return pl.pallas_call(
        body,
        grid=(n_pad // blk,),
        in_specs=[
            pl.BlockSpec((NC, blk, d), lambda i: (0, i, 0)),
            pl.BlockSpec((blk, d), lambda i: (i, 0)),
            pl.BlockSpec((NC, blk), lambda i: (0, i)),
            pl.BlockSpec((1, d), lambda i: (0, 0)),
        ],
        out_specs=pl.BlockSpec((blk, d), lambda i: (i, 0)),
        out_shape=jax.ShapeDtypeStruct((n, d), jnp.float32),
    )(accp, g, degp, b2)


def kernel(x, edge_index, W, b):
    n, d = x.shape
    e = edge_index.shape[1]
    n_pad = ((n + NS * CH - 1) // (NS * CH)) * (NS * CH)

    nrows = e // CH
    base = nrows // NW
    rem = nrows - base * NW

    degp = _sc_hist(edge_index, n_pad, base, rem)

    blk = 5120
    h = _tc_matmul(x, W, n_pad, blk)

    g = _tc_scale(h, degp, n_pad, blk)

    accp = _sc_gather_scatter(g, edge_index, n_pad, d, base, rem)

    return _tc_finalize(accp, g, degp, b.reshape(1, d), n, n_pad, blk)
